# baseline (speedup 1.0000x reference)
import functools

import jax
import jax.numpy as jnp
from jax import lax
from jax.experimental import pallas as pl
from jax.experimental.pallas import tpu as pltpu
from jax.experimental.pallas import tpu_sc as plsc

BATCH = 16384
EMB = 64
PAIR = 2 * EMB
HID = 128
NW = 32
BPW = BATCH // NW
CHUNK = 128
NCHUNK = BPW // CHUNK
NBUF = 2


def _sc_gather(dom2, go2, Wd2, Wg2):
    mesh = plsc.VectorSubcoreMesh(core_axis_name="c", subcore_axis_name="s")

    @functools.partial(
        pl.kernel,
        mesh=mesh,
        out_type=(
            jax.ShapeDtypeStruct((BATCH, PAIR), jnp.float32),
            jax.ShapeDtypeStruct((BATCH, PAIR), jnp.float32),
        ),
        scratch_types=(
            pltpu.VMEM((BPW,), jnp.int32),
            pltpu.VMEM((BPW,), jnp.int32),
            pltpu.VMEM((NBUF, CHUNK, PAIR), jnp.float32),
            pltpu.VMEM((NBUF, CHUNK, PAIR), jnp.float32),
            pltpu.SemaphoreType.DMA,
            pltpu.SemaphoreType.DMA,
            pltpu.SemaphoreType.DMA,
            pltpu.SemaphoreType.DMA,
            pltpu.SemaphoreType.DMA,
            pltpu.SemaphoreType.DMA,
        ),
    )
    def gather_kernel(dom_hbm, go_hbm, wd_hbm, wg_hbm, outd_hbm, outg_hbm,
                      idx_d, idx_g, rows_d, rows_g,
                      gsem_d, gsem_g, wd0, wd1, wg0, wg1):
        wid = lax.axis_index("s") * 2 + lax.axis_index("c")
        base = wid * BPW
        pltpu.sync_copy(dom_hbm.at[pl.ds(base, BPW)], idx_d)
        pltpu.sync_copy(go_hbm.at[pl.ds(base, BPW)], idx_g)
        wsems_d = (wd0, wd1)
        wsems_g = (wg0, wg1)
        pending = [None] * NBUF
        for c in range(NCHUNK):
            b = c % NBUF
            if pending[b] is not None:
                pending[b][0].wait()
                pending[b][1].wait()
            sl = pl.ds(c * CHUNK, CHUNK)
            gd = pltpu.async_copy(wd_hbm.at[idx_d.at[sl]], rows_d.at[b], gsem_d)
            gg = pltpu.async_copy(wg_hbm.at[idx_g.at[sl]], rows_g.at[b], gsem_g)
            gd.wait()
            gg.wait()
            osl = pl.ds(base + c * CHUNK, CHUNK)
            pending[b] = (
                pltpu.async_copy(rows_d.at[b], outd_hbm.at[osl], wsems_d[b]),
                pltpu.async_copy(rows_g.at[b], outg_hbm.at[osl], wsems_g[b]),
            )
        for p in pending:
            if p is not None:
                p[0].wait()
                p[1].wait()

    return gather_kernel(dom2, go2, Wd2, Wg2)


def _head_body(dp_ref, gp_ref, dpar_ref, gpar_ref,
               w1_ref, b1_ref, w2_ref, b2_ref, o_ref):
    dp = dp_ref[...]
    gp = gp_ref[...]
    d = jnp.where(dpar_ref[...] == 1, dp[:, EMB:], dp[:, :EMB])
    g = jnp.where(gpar_ref[...] == 1, gp[:, EMB:], gp[:, :EMB])
    nd = jnp.sqrt(jnp.sum(d * d, axis=1, keepdims=True))
    ng = jnp.sqrt(jnp.sum(g * g, axis=1, keepdims=True))
    sd = jnp.where(nd > 1.0, 1.0 / (nd + 1e-7), 1.0)
    sg = jnp.where(ng > 1.0, 1.0 / (ng + 1e-7), 1.0)
    feat = (d * sd) * (g * sg)
    h = jnp.maximum(
        jnp.dot(feat, w1_ref[...], preferred_element_type=jnp.float32) + b1_ref[...],
        0.0,
    )
    o = jnp.sum(h * w2_ref[...], axis=1) + b2_ref[0, 0]
    o_ref[...] = jax.nn.sigmoid(o)


def _tc_head(d_pair, g_pair, d_par, g_par, W1, b1, W2, b2, blk=2048):
    nblk = BATCH // blk
    out = pl.pallas_call(
        _head_body,
        grid=(nblk,),
        in_specs=[
            pl.BlockSpec((blk, PAIR), lambda i: (i, 0)),
            pl.BlockSpec((blk, PAIR), lambda i: (i, 0)),
            pl.BlockSpec((blk, 1), lambda i: (i, 0)),
            pl.BlockSpec((blk, 1), lambda i: (i, 0)),
            pl.BlockSpec((EMB, HID), lambda i: (0, 0)),
            pl.BlockSpec((1, HID), lambda i: (0, 0)),
            pl.BlockSpec((1, HID), lambda i: (0, 0)),
            pl.BlockSpec((1, 1), lambda i: (0, 0)),
        ],
        out_specs=pl.BlockSpec((blk,), lambda i: (i,)),
        out_shape=jax.ShapeDtypeStruct((BATCH,), jnp.float32),
    )(d_pair, g_pair, d_par, g_par,
      W1, b1.reshape(1, HID), W2.reshape(1, HID), b2.reshape(1, 1))
    return out


def kernel(domain_id, go_id, W_domain, W_go, W1, b1, W2, b2):
    Wd2 = W_domain.reshape(W_domain.shape[0] // 2, PAIR)
    Wg2 = W_go.reshape(W_go.shape[0] // 2, PAIR)
    dom2 = jax.lax.shift_right_logical(domain_id, 1)
    go2 = jax.lax.shift_right_logical(go_id, 1)
    d_par = jax.lax.bitwise_and(domain_id, 1).reshape(BATCH, 1)
    g_par = jax.lax.bitwise_and(go_id, 1).reshape(BATCH, 1)
    d_pair, g_pair = _sc_gather(dom2, go2, Wd2, Wg2)
    return _tc_head(d_pair, g_pair, d_par, g_par, W1, b1, W2, b2)


# speedup vs baseline: 2.0510x; 2.0510x over previous
import functools

import jax
import jax.numpy as jnp
from jax import lax
from jax.experimental import pallas as pl
from jax.experimental.pallas import tpu as pltpu
from jax.experimental.pallas import tpu_sc as plsc

BATCH = 16384
EMB = 64
PAIR = 2 * EMB
HID = 128
NW = 32
BPW = BATCH // NW
CHUNK = 128
NCHUNK = BPW // CHUNK
NBUF = 2
LBLK = 8192


def _tp_body(in_ref, o_ref):
    x = in_ref[...]
    a = x[:, :LBLK]
    b = x[:, LBLK:]
    o_ref[...] = jnp.concatenate([a.T, b.T], axis=1)


def _tc_transpose_pair(wt):
    v = wt.shape[1]
    grid = (v + 2 * LBLK - 1) // (2 * LBLK)
    return pl.pallas_call(
        _tp_body,
        grid=(grid,),
        in_specs=[pl.BlockSpec((EMB, 2 * LBLK), lambda i: (0, i))],
        out_specs=pl.BlockSpec((LBLK, PAIR), lambda i: (i, 0)),
        out_shape=jax.ShapeDtypeStruct((grid * LBLK, PAIR), jnp.float32),
    )(wt)


def _sc_gather(dom2, go2, Wd2, Wg2):
    mesh = plsc.VectorSubcoreMesh(core_axis_name="c", subcore_axis_name="s")

    @functools.partial(
        pl.kernel,
        mesh=mesh,
        out_type=(
            jax.ShapeDtypeStruct((BATCH, PAIR), jnp.float32),
            jax.ShapeDtypeStruct((BATCH, PAIR), jnp.float32),
        ),
        scratch_types=(
            pltpu.VMEM((BPW,), jnp.int32),
            pltpu.VMEM((BPW,), jnp.int32),
            pltpu.VMEM((NBUF, CHUNK, PAIR), jnp.float32),
            pltpu.VMEM((NBUF, CHUNK, PAIR), jnp.float32),
            pltpu.SemaphoreType.DMA,
            pltpu.SemaphoreType.DMA,
            pltpu.SemaphoreType.DMA,
            pltpu.SemaphoreType.DMA,
            pltpu.SemaphoreType.DMA,
            pltpu.SemaphoreType.DMA,
        ),
    )
    def gather_kernel(dom_hbm, go_hbm, wd_hbm, wg_hbm, outd_hbm, outg_hbm,
                      idx_d, idx_g, rows_d, rows_g,
                      gsem_d, gsem_g, wd0, wd1, wg0, wg1):
        wid = lax.axis_index("s") * 2 + lax.axis_index("c")
        base = wid * BPW
        pltpu.sync_copy(dom_hbm.at[pl.ds(base, BPW)], idx_d)
        pltpu.sync_copy(go_hbm.at[pl.ds(base, BPW)], idx_g)
        wsems_d = (wd0, wd1)
        wsems_g = (wg0, wg1)
        pending = [None] * NBUF
        for c in range(NCHUNK):
            b = c % NBUF
            if pending[b] is not None:
                pending[b][0].wait()
                pending[b][1].wait()
            sl = pl.ds(c * CHUNK, CHUNK)
            gd = pltpu.async_copy(wd_hbm.at[idx_d.at[sl]], rows_d.at[b], gsem_d)
            gg = pltpu.async_copy(wg_hbm.at[idx_g.at[sl]], rows_g.at[b], gsem_g)
            gd.wait()
            gg.wait()
            osl = pl.ds(base + c * CHUNK, CHUNK)
            pending[b] = (
                pltpu.async_copy(rows_d.at[b], outd_hbm.at[osl], wsems_d[b]),
                pltpu.async_copy(rows_g.at[b], outg_hbm.at[osl], wsems_g[b]),
            )
        for p in pending:
            if p is not None:
                p[0].wait()
                p[1].wait()

    return gather_kernel(dom2, go2, Wd2, Wg2)


def _head_body(dp_ref, gp_ref, dpar_ref, gpar_ref,
               w1_ref, b1_ref, w2_ref, b2_ref, o_ref):
    dp = dp_ref[...]
    gp = gp_ref[...]
    d = jnp.where(dpar_ref[...] == 1, dp[:, EMB:], dp[:, :EMB])
    g = jnp.where(gpar_ref[...] == 1, gp[:, EMB:], gp[:, :EMB])
    nd = jnp.sqrt(jnp.sum(d * d, axis=1, keepdims=True))
    ng = jnp.sqrt(jnp.sum(g * g, axis=1, keepdims=True))
    sd = jnp.where(nd > 1.0, 1.0 / (nd + 1e-7), 1.0)
    sg = jnp.where(ng > 1.0, 1.0 / (ng + 1e-7), 1.0)
    feat = (d * sd) * (g * sg)
    h = jnp.maximum(
        jnp.dot(feat, w1_ref[...], preferred_element_type=jnp.float32) + b1_ref[...],
        0.0,
    )
    o = jnp.sum(h * w2_ref[...], axis=1) + b2_ref[0, 0]
    o_ref[...] = jax.nn.sigmoid(o)


def _tc_head(d_pair, g_pair, d_par, g_par, W1, b1, W2, b2, blk=2048):
    nblk = BATCH // blk
    out = pl.pallas_call(
        _head_body,
        grid=(nblk,),
        in_specs=[
            pl.BlockSpec((blk, PAIR), lambda i: (i, 0)),
            pl.BlockSpec((blk, PAIR), lambda i: (i, 0)),
            pl.BlockSpec((blk, 1), lambda i: (i, 0)),
            pl.BlockSpec((blk, 1), lambda i: (i, 0)),
            pl.BlockSpec((EMB, HID), lambda i: (0, 0)),
            pl.BlockSpec((1, HID), lambda i: (0, 0)),
            pl.BlockSpec((1, HID), lambda i: (0, 0)),
            pl.BlockSpec((1, 1), lambda i: (0, 0)),
        ],
        out_specs=pl.BlockSpec((blk,), lambda i: (i,)),
        out_shape=jax.ShapeDtypeStruct((BATCH,), jnp.float32),
    )(d_pair, g_pair, d_par, g_par,
      W1, b1.reshape(1, HID), W2.reshape(1, HID), b2.reshape(1, 1))
    return out


def kernel(domain_id, go_id, W_domain, W_go, W1, b1, W2, b2):
    Wd2 = _tc_transpose_pair(W_domain.T)
    Wg2 = W_go.reshape(W_go.shape[0] // 2, PAIR)
    dom2 = (jax.lax.shift_right_logical(domain_id, 14) * LBLK
            + jax.lax.bitwise_and(domain_id, LBLK - 1))
    go2 = jax.lax.shift_right_logical(go_id, 1)
    d_par = jax.lax.bitwise_and(
        jax.lax.shift_right_logical(domain_id, 13), 1).reshape(BATCH, 1)
    g_par = jax.lax.bitwise_and(go_id, 1).reshape(BATCH, 1)
    d_pair, g_pair = _sc_gather(dom2, go2, Wd2, Wg2)
    return _tc_head(d_pair, g_pair, d_par, g_par, W1, b1, W2, b2)
